# baseline (speedup 1.0000x reference)
import functools

import jax
import jax.numpy as jnp
from jax import lax
from jax.experimental import pallas as pl
from jax.experimental.pallas import tpu as pltpu
from jax.experimental.pallas import tpu_sc as plsc

D_MODEL = 64
SCALE = 8.0

NC, NS = 2, 16
NW = NC * NS
B_TOTAL = 4096 * 200
PER_W = B_TOTAL // NW
C = 640
G = 128
K = C // G
NCHUNK = PER_W // C
XROW_W = PER_W // G


def _emb_body(x_hbm, tab_hbm, out_hbm, idx_v, rows_v,
              gsem0, gsem1, osem0, osem1):
    gsem = (gsem0, gsem1)
    osem = (osem0, osem1)
    wid = lax.axis_index("s") * NC + lax.axis_index("c")
    base = wid * PER_W
    xrow0 = wid * XROW_W

    pltpu.sync_copy(x_hbm.at[pl.ds(xrow0, XROW_W)], idx_v)

    def fetch(g, b):
        for j in range(K):
            pltpu.async_copy(tab_hbm.at[idx_v.at[g * K + j]],
                             rows_v.at[b, pl.ds(j * G, G), :], gsem[b])

    def drain(sem, b):
        pltpu.make_async_copy(out_hbm.at[pl.ds(0, C), :],
                              rows_v.at[b], sem).wait()

    def scale(b):
        @pl.loop(0, C)
        def _row(i):
            for c4 in range(D_MODEL // 16):
                sl = pl.ds(c4 * 16, 16)
                rows_v[b, i, sl] = rows_v[b, i, sl] * SCALE

    def out_copy(g, b):
        pltpu.async_copy(rows_v.at[b],
                         out_hbm.at[pl.ds(base + g * C, C), :], osem[b])

    fetch(0, 0)
    fetch(1, 1)
    drain(gsem[0], 0)
    scale(0)
    out_copy(0, 0)

    @pl.loop(1, NCHUNK - 1, step=2)
    def _steady(gg):
        for db in range(2):
            g = gg + db
            b = (1 + db) % 2
            drain(osem[1 - b], 1 - b)
            fetch(g + 1, 1 - b)
            drain(gsem[b], b)
            scale(b)
            out_copy(g, b)

    drain(osem[0], 0)
    drain(gsem[1], 1)
    scale(1)
    out_copy(NCHUNK - 1, 1)
    drain(osem[1], 1)


@functools.partial(jax.jit, static_argnames=())
def _run(x2d, table):
    mesh = plsc.VectorSubcoreMesh(core_axis_name="c", subcore_axis_name="s")
    f = functools.partial(
        pl.kernel,
        mesh=mesh,
        compiler_params=pltpu.CompilerParams(use_tc_tiling_on_sc=False),
        out_type=jax.ShapeDtypeStruct((B_TOTAL, D_MODEL), jnp.float32),
        scratch_types=[
            pltpu.VMEM((XROW_W, G), jnp.int32),
            pltpu.VMEM((2, C, D_MODEL), jnp.float32),
            pltpu.SemaphoreType.DMA,
            pltpu.SemaphoreType.DMA,
            pltpu.SemaphoreType.DMA,
            pltpu.SemaphoreType.DMA,
        ],
    )(_emb_body)
    return f(x2d, table)


def kernel(x, table):
    x2d = x.reshape(B_TOTAL // G, G).astype(jnp.int32)
    out = _run(x2d, table)
    return out.reshape(x.shape[0], x.shape[1], D_MODEL)


# speedup vs baseline: 1.0856x; 1.0856x over previous
import functools

import jax
import jax.numpy as jnp
from jax import lax
from jax.experimental import pallas as pl
from jax.experimental.pallas import tpu as pltpu
from jax.experimental.pallas import tpu_sc as plsc

D_MODEL = 64
SCALE = 8.0

NC, NS = 2, 16
NW = NC * NS
N_TOK = 4096
N_POS = 200
L = 128
NBLK = N_TOK // L
PAD = 129


def _emb_body(x_hbm, tab_hbm, out_hbm, idx_v, rows_v, obuf,
              gsem0, gsem1, osem0, osem1):
    gsem = (gsem0, gsem1)
    osem = (osem0, osem1)
    wid = lax.axis_index("s") * NC + lax.axis_index("c")

    pltpu.sync_copy(x_hbm.at[:, pl.ds(wid * L, L)], idx_v)

    lane = lax.iota(jnp.int32, 16)
    a_idx = [(lane + 16 * c) >> 3 for c in range(4)]
    s_idx = [(lane + 16 * c) & 7 for c in range(4)]

    def fetch(j, b):
        pltpu.async_copy(tab_hbm.at[idx_v.at[j]], rows_v.at[b], gsem[b])

    def drain_g(b):
        pltpu.make_async_copy(tab_hbm.at[pl.ds(0, L)], rows_v.at[b],
                              gsem[b]).wait()

    def drain_o(b):
        pltpu.make_async_copy(out_hbm.at[0, :, 0],
                              obuf.at[b, :, :, pl.ds(0, L)], osem[b]).wait()

    def transpose_scale(b):
        @pl.loop(0, L)
        def _r(r):
            l_idx = jnp.full((16,), r, dtype=jnp.int32)
            for c in range(4):
                v = rows_v[b, r, pl.ds(16 * c, 16)] * SCALE
                plsc.store_scatter(obuf.at[b], [a_idx[c], s_idx[c], l_idx], v)

    def out_copy(j, b):
        pltpu.async_copy(obuf.at[b, :, :, pl.ds(0, L)],
                         out_hbm.at[j, :, wid], osem[b])

    fetch(0, 0)
    fetch(1, 1)
    drain_g(0)
    transpose_scale(0)
    out_copy(0, 0)

    @pl.loop(1, N_POS - 1, step=2)
    def _steady(jj):
        for db in range(2):
            j = jj + db
            b = (1 + db) % 2
            drain_o(1 - b)
            fetch(j + 1, 1 - b)
            drain_g(b)
            transpose_scale(b)
            out_copy(j, b)

    drain_o(0)
    drain_g(1)
    transpose_scale(1)
    out_copy(N_POS - 1, 1)
    drain_o(1)


@jax.jit
def _run(x_t, table):
    mesh = plsc.VectorSubcoreMesh(core_axis_name="c", subcore_axis_name="s")
    f = functools.partial(
        pl.kernel,
        mesh=mesh,
        compiler_params=pltpu.CompilerParams(use_tc_tiling_on_sc=False,
                                             needs_layout_passes=False),
        out_type=jax.ShapeDtypeStruct((N_POS, D_MODEL // 8, NBLK, 8, L),
                                      jnp.float32),
        scratch_types=[
            pltpu.VMEM((N_POS, L), jnp.int32),
            pltpu.VMEM((2, L, D_MODEL), jnp.float32),
            pltpu.VMEM((2, D_MODEL // 8, 8, PAD), jnp.float32),
            pltpu.SemaphoreType.DMA,
            pltpu.SemaphoreType.DMA,
            pltpu.SemaphoreType.DMA,
            pltpu.SemaphoreType.DMA,
        ],
    )(_emb_body)
    return f(x_t, table)


def kernel(x, table):
    x_t = x.astype(jnp.int32).T
    out5 = _run(x_t, table)
    out = jnp.transpose(out5, (2, 4, 0, 1, 3)).reshape(N_TOK, N_POS, D_MODEL)
    return out
